# baseline (speedup 1.0000x reference)
import functools

import jax
import jax.numpy as jnp
from jax.experimental import pallas as pl
from jax.experimental.pallas import tpu as pltpu

_CHUNK = 16
_GROUP = 128
_PRUNE_ZEROED = 96.0
_QB_MAX = 127.0
_QB_MIN = -128.0
_QD_MAX = 7.0
_QD_MIN = -8.0
_EPS = 1e-5


def _body(dl_ref, x_ref, o_ref):
    B = x_ref.shape[2]
    x = x_ref[0, 0]
    dl = dl_ref[0]
    pid = pl.program_id(1)
    row0 = (pid // 2 + (pid % 2) * 2) * B

    @pl.when(row0 >= dl)
    def _copy():
        o_ref[0, 0] = x

    @pl.when(row0 < dl)
    def _quant():
        nc = B // _CHUNK
        x3 = x.reshape(nc, _CHUNK, _GROUP)
        xb = x3[:, 0, :]
        sb = jnp.maximum(jnp.max(xb, axis=1, keepdims=True) / _QB_MAX, _EPS)
        qb = jnp.maximum(jnp.round(xb / sb), _QB_MIN) * sb
        qbb = jnp.broadcast_to(qb[:, None, :], (nc, _CHUNK, _GROUP)
                               ).reshape(B, _GROUP)
        ri = jax.lax.broadcasted_iota(jnp.int32, (B, 1), 0)
        notbase = ri % _CHUNK != 0
        d = jnp.where(notbase, x - qbb, 0.0)
        dt = d.T
        sd = jnp.maximum(jnp.max(dt, axis=0, keepdims=True) / _QD_MAX, _EPS)
        di = jnp.maximum(jnp.round(dt / sd), _QD_MIN)
        dq = di * sd
        m = jnp.abs(di)
        mid = jnp.full((1, B), 7.0, jnp.float32)
        for step in (4.0, 2.0, 1.0):
            cnt = jnp.sum(jnp.where(m <= mid, 1.0, 0.0), axis=0,
                          keepdims=True)
            mid = mid + jnp.where(cnt > _PRUNE_ZEROED, -step, step)
        cnt = jnp.sum(jnp.where(m <= mid, 1.0, 0.0), axis=0, keepdims=True)
        tval = jnp.where(cnt > _PRUNE_ZEROED, mid, mid + 1.0)
        mlt = m < tval
        e_t = m == tval
        G = jnp.where(mlt, 1.0, 0.0)
        E = jnp.where(e_t, 1.0, 0.0)
        jr = jax.lax.broadcasted_iota(jnp.int32, (2 * _GROUP, _GROUP), 0)
        ic = jax.lax.broadcasted_iota(jnp.int32, (2 * _GROUP, _GROUP), 1)
        W = ((jr < _GROUP) | (jr - _GROUP < ic)).astype(jnp.float32)
        rank = jax.lax.dot_general(W, jnp.concatenate([G, E], axis=0),
                                   (((0,), (0,)), ((), ())),
                                   preferred_element_type=jnp.float32)
        zero = mlt | (e_t & (rank < _PRUNE_ZEROED))
        dqp = jnp.where(zero, 0.0, dq).T
        outq = qbb + dqp
        out_rows = (row0 + ri) < dl
        o_ref[0, 0] = jnp.where(out_rows, outq, x)


@functools.partial(jax.jit, static_argnames=("interpret",))
def _run(feat, dl_arr, interpret=False):
    _, H, T, D = feat.shape
    B = 1024
    grid = (H, T // B)
    return pl.pallas_call(
        _body,
        grid=grid,
        in_specs=[
            pl.BlockSpec(memory_space=pltpu.SMEM),
            pl.BlockSpec((1, 1, B, D),
                         lambda h, tb: (0, h, tb // 2 + (tb % 2) * 2, 0)),
        ],
        out_specs=pl.BlockSpec((1, 1, B, D),
                               lambda h, tb: (0, h, tb // 2 + (tb % 2) * 2, 0)),
        out_shape=jax.ShapeDtypeStruct(feat.shape, feat.dtype),
        interpret=interpret,
    )(dl_arr, feat)


def kernel(feat, diff_len):
    dl_arr = jnp.asarray(diff_len, jnp.int32).reshape(1)
    return _run(feat, dl_arr)


# speedup vs baseline: 1.2380x; 1.2380x over previous
import functools

import jax
import jax.numpy as jnp
from jax.experimental import pallas as pl
from jax.experimental.pallas import tpu as pltpu

_CHUNK = 16
_GROUP = 128
_PRUNE_ZEROED = 96.0
_QB_MAX = 127.0
_QB_MIN = -128.0
_QD_MAX = 7.0
_QD_MIN = -8.0
_EPS = 1e-5


def _body(dl_ref, x_ref, o_ref):
    B = x_ref.shape[2]
    x = x_ref[0, 0]
    dl = dl_ref[0]
    row0 = pl.program_id(1) * B

    @pl.when(row0 >= dl)
    def _copy():
        o_ref[0, 0] = x

    @pl.when(row0 < dl)
    def _quant():
        nc = B // _CHUNK
        x3 = x.reshape(nc, _CHUNK, _GROUP)
        xb = x3[:, 0, :]
        sb = jnp.maximum(jnp.max(xb, axis=1, keepdims=True) / _QB_MAX, _EPS)
        qb = jnp.maximum(jnp.round(xb / sb), _QB_MIN) * sb
        qbb = jnp.broadcast_to(qb[:, None, :], (nc, _CHUNK, _GROUP)
                               ).reshape(B, _GROUP)
        ri = jax.lax.broadcasted_iota(jnp.int32, (B, 1), 0)
        notbase = ri % _CHUNK != 0
        d = jnp.where(notbase, x - qbb, 0.0)
        dt = d.T
        sd = jnp.maximum(jnp.max(dt, axis=0, keepdims=True) / _QD_MAX, _EPS)
        di = jnp.maximum(jnp.round(dt / sd), _QD_MIN)
        dq = di * sd
        m = jnp.abs(di)
        mid = jnp.full((1, B), 7.0, jnp.float32)
        for step in (4.0, 2.0, 1.0):
            cnt = jnp.sum(jnp.where(m <= mid, 1.0, 0.0), axis=0,
                          keepdims=True)
            mid = mid + jnp.where(cnt > _PRUNE_ZEROED, -step, step)
        cnt = jnp.sum(jnp.where(m <= mid, 1.0, 0.0), axis=0, keepdims=True)
        tval = jnp.where(cnt > _PRUNE_ZEROED, mid, mid + 1.0)
        mlt = m < tval
        e_t = m == tval
        G = jnp.where(mlt, 1.0, 0.0)
        E = jnp.where(e_t, 1.0, 0.0)
        jr = jax.lax.broadcasted_iota(jnp.int32, (2 * _GROUP, _GROUP), 0)
        ic = jax.lax.broadcasted_iota(jnp.int32, (2 * _GROUP, _GROUP), 1)
        W = ((jr < _GROUP) | (jr - _GROUP < ic)).astype(jnp.float32)
        rank = jax.lax.dot_general(W, jnp.concatenate([G, E], axis=0),
                                   (((0,), (0,)), ((), ())),
                                   preferred_element_type=jnp.float32)
        zero = mlt | (e_t & (rank < _PRUNE_ZEROED))
        dqp = jnp.where(zero, 0.0, dq).T
        outq = qbb + dqp
        out_rows = (row0 + ri) < dl
        o_ref[0, 0] = jnp.where(out_rows, outq, x)


@functools.partial(jax.jit, static_argnames=("interpret",))
def _run(feat, dl_arr, interpret=False):
    _, H, T, D = feat.shape
    B = 1024
    grid = (H, T // B)
    return pl.pallas_call(
        _body,
        grid=grid,
        in_specs=[
            pl.BlockSpec(memory_space=pltpu.SMEM),
            pl.BlockSpec((1, 1, B, D), lambda h, tb: (0, h, tb, 0)),
        ],
        out_specs=pl.BlockSpec((1, 1, B, D), lambda h, tb: (0, h, tb, 0)),
        out_shape=jax.ShapeDtypeStruct(feat.shape, feat.dtype),
        interpret=interpret,
    )(dl_arr, feat)


def kernel(feat, diff_len):
    dl_arr = jnp.asarray(diff_len, jnp.int32).reshape(1)
    return _run(feat, dl_arr)
